# baseline (speedup 1.0000x reference)
import functools

import jax
import jax.numpy as jnp
from jax import lax
from jax.experimental import pallas as pl
from jax.experimental.pallas import tpu as pltpu
from jax.experimental.pallas import tpu_sc as plsc

B = 16384
HALF = B // 2
DIM = 64
MARGIN = 1.0
NC = 2
NS = 16
NW = NC * NS
CHUNK = HALF // NW
L = 16
NGROUPS = CHUNK // L


def _rsqrt_newton(x):
    bits = lax.bitcast_convert_type(x, jnp.int32)
    y = lax.bitcast_convert_type(
        jnp.int32(0x5F3759DF) - lax.shift_right_logical(bits, 1), jnp.float32)
    half_x = 0.5 * x
    for _ in range(3):
        y = y * (1.5 - half_x * y * y)
    return y


def _chunk_scores(hbuf, tbuf, rbuf, sbuf):
    def g_body(g, carry):
        row_ids = g * L + lax.iota(jnp.int32, L)

        def j_body(j, acc):
            h2, t2, r2, hr, tr, ht = acc
            jv = jnp.full((L,), j, dtype=jnp.int32)
            hv = plsc.load_gather(hbuf, [row_ids, jv])
            tv = plsc.load_gather(tbuf, [row_ids, jv])
            rv = plsc.load_gather(rbuf, [row_ids, jv])
            return (h2 + hv * hv, t2 + tv * tv, r2 + rv * rv,
                    hr + hv * rv, tr + tv * rv, ht + hv * tv)

        zeros = jnp.zeros((L,), jnp.float32)
        h2, t2, r2, hr, tr, ht = lax.fori_loop(
            0, DIM, j_body, (zeros,) * 6, unroll=8)

        sh = jnp.minimum(1.0, _rsqrt_newton(h2))
        st = jnp.minimum(1.0, _rsqrt_newton(t2))
        dsq = (sh * sh * h2 + r2 + st * st * t2
               + 2.0 * sh * hr - 2.0 * st * tr - 2.0 * (sh * st) * ht)
        dsq = jnp.maximum(dsq, 0.0)
        sbuf[pl.ds(g * L, L)] = dsq * _rsqrt_newton(dsq)
        return carry

    lax.fori_loop(0, NGROUPS, g_body, 0)


def _transe_body(bh_hbm, bt_hbm, br_hbm, ent_hbm, rel_hbm,
                 pos_out, neg_out, part_out,
                 ihp, itp, irp, ihn, itn, irn,
                 hp, tp, rp, hn, tn, rn,
                 psb, nsb, pvec, sem_p, sem_n):
    wid = lax.axis_index("s") * NC + lax.axis_index("c")
    pbase = wid * CHUNK
    nbase = HALF + wid * CHUNK

    pltpu.sync_copy(bh_hbm.at[pl.ds(pbase, CHUNK)], ihp)
    pltpu.sync_copy(bt_hbm.at[pl.ds(pbase, CHUNK)], itp)
    pltpu.sync_copy(br_hbm.at[pl.ds(pbase, CHUNK)], irp)
    pltpu.sync_copy(bh_hbm.at[pl.ds(nbase, CHUNK)], ihn)
    pltpu.sync_copy(bt_hbm.at[pl.ds(nbase, CHUNK)], itn)
    pltpu.sync_copy(br_hbm.at[pl.ds(nbase, CHUNK)], irn)

    cp1 = pltpu.async_copy(ent_hbm.at[ihp], hp, sem_p)
    cp2 = pltpu.async_copy(ent_hbm.at[itp], tp, sem_p)
    cp3 = pltpu.async_copy(rel_hbm.at[irp], rp, sem_p)
    cn1 = pltpu.async_copy(ent_hbm.at[ihn], hn, sem_n)
    cn2 = pltpu.async_copy(ent_hbm.at[itn], tn, sem_n)
    cn3 = pltpu.async_copy(rel_hbm.at[irn], rn, sem_n)

    cp1.wait()
    cp2.wait()
    cp3.wait()
    _chunk_scores(hp, tp, rp, psb)

    cn1.wait()
    cn2.wait()
    cn3.wait()
    _chunk_scores(hn, tn, rn, nsb)

    def l_body(g, acc):
        p = psb[pl.ds(g * L, L)]
        n = nsb[pl.ds(g * L, L)]
        return acc + jnp.maximum(0.0, p - n + MARGIN)

    lacc = lax.fori_loop(0, NGROUPS, l_body, jnp.zeros((L,), jnp.float32))
    pvec[...] = jnp.full((L,), jnp.sum(lacc), jnp.float32)

    pltpu.sync_copy(psb, pos_out.at[pl.ds(pbase, CHUNK)])
    pltpu.sync_copy(nsb, neg_out.at[pl.ds(pbase, CHUNK)])
    pltpu.sync_copy(pvec, part_out.at[wid])


@jax.jit
def _transe_sc(bh, bt, br, ent_emb, rel_emb):
    mesh = plsc.VectorSubcoreMesh(
        core_axis_name="c", subcore_axis_name="s",
        num_cores=NC, num_subcores=NS)
    f = pl.kernel(
        _transe_body,
        out_type=(
            jax.ShapeDtypeStruct((HALF,), jnp.float32),
            jax.ShapeDtypeStruct((HALF,), jnp.float32),
            jax.ShapeDtypeStruct((NW, L), jnp.float32),
        ),
        mesh=mesh,
        compiler_params=pltpu.CompilerParams(
            needs_layout_passes=False, use_tc_tiling_on_sc=False),
        scratch_types=(
            [pltpu.VMEM((CHUNK,), jnp.int32) for _ in range(6)]
            + [pltpu.VMEM((CHUNK, DIM), jnp.float32) for _ in range(6)]
            + [pltpu.VMEM((CHUNK,), jnp.float32) for _ in range(2)]
            + [pltpu.VMEM((L,), jnp.float32),
               pltpu.SemaphoreType.DMA, pltpu.SemaphoreType.DMA]
        ),
    )
    return f(bh, bt, br, ent_emb, rel_emb)


def kernel(batch_h, batch_t, batch_r, batch_y, ent_emb, rel_emb):
    del batch_y
    bh = batch_h.astype(jnp.int32)
    bt = batch_t.astype(jnp.int32)
    br = batch_r.astype(jnp.int32)
    pos_score, neg_score, partials = _transe_sc(bh, bt, br, ent_emb, rel_emb)
    loss = jnp.sum(partials[:, 0])
    return (loss, pos_score, neg_score)
